# baseline (speedup 1.0000x reference)
import functools

import jax
import jax.numpy as jnp
from jax import lax
from jax.experimental import pallas as pl
from jax.experimental.pallas import tpu as pltpu
from jax.experimental.pallas import tpu_sc as plsc

E = 64
K = 8
L = 16
NW = 32


def _logits_body(wt_ref, x_ref, outt_ref):
    outt_ref[0] = lax.dot_general(
        wt_ref[...], x_ref[...],
        dimension_numbers=(((1,), (1,)), ((), ())),
        preferred_element_type=jnp.float32,
    )


def _router_logits_t(x2d, wt, b, s):
    t, h = x2d.shape
    bt = 1024
    nblk = t // bt
    bpb = s // bt
    return pl.pallas_call(
        _logits_body,
        grid=(nblk,),
        in_specs=[
            pl.BlockSpec((E, h), lambda i: (0, 0)),
            pl.BlockSpec((bt, h), lambda i: (i, 0)),
        ],
        out_specs=pl.BlockSpec((1, E, bt), lambda i: (i // bpb, 0, i % bpb)),
        out_shape=jax.ShapeDtypeStruct((b, E, s), jnp.float32),
    )(wt, x2d)


def _make_topk_sc(b, s, interpret=False):
    nc, ns = 2, 16
    t = b * s
    tpw = t // NW
    ncol = tpw // L
    wpb = s // tpw

    mesh = plsc.VectorSubcoreMesh(core_axis_name="c", subcore_axis_name="s",
                                  num_cores=nc, num_subcores=ns)

    @functools.partial(
        pl.kernel,
        out_type=(
            jax.ShapeDtypeStruct((b, K, s), jnp.float32),
            jax.ShapeDtypeStruct((b, K, s), jnp.int32),
        ),
        mesh=mesh,
        scratch_types=[
            pltpu.VMEM((E, tpw), jnp.float32),
            pltpu.VMEM((K, tpw), jnp.float32),
            pltpu.VMEM((K, tpw), jnp.int32),
        ],
        compiler_params=pltpu.CompilerParams(needs_layout_passes=False),
        interpret=interpret,
    )
    def topk(logitst_hbm, w_hbm, i_hbm, loc, wloc, iloc):
        wid = lax.axis_index("s") * nc + lax.axis_index("c")
        bi = wid // wpb
        off = (wid % wpb) * tpw
        pltpu.sync_copy(logitst_hbm.at[bi, :, pl.ds(off, tpw)], loc)

        neg_inf = jnp.full((L,), -jnp.inf, jnp.float32)
        zero_i = jnp.zeros((L,), jnp.int32)
        ncc = 2

        def col_pair(cp, _):
            c0 = cp * ncc

            def body(e, carry):
                vs = [list(v) for v in carry[0]]
                ix = [list(i) for i in carry[1]]
                eidx = jnp.full((L,), e, jnp.int32)
                for j in range(ncc):
                    cur_v = loc[e, pl.ds((c0 + j) * L, L)]
                    cur_i = eidx
                    for kk in range(K):
                        m = cur_v > vs[j][kk]
                        nv = jnp.maximum(cur_v, vs[j][kk])
                        lo = jnp.minimum(cur_v, vs[j][kk])
                        ni = jnp.where(m, cur_i, ix[j][kk])
                        cur_i = jnp.where(m, ix[j][kk], cur_i)
                        vs[j][kk] = nv
                        ix[j][kk] = ni
                        cur_v = lo
                    del cur_v, cur_i
                return (tuple(tuple(v) for v in vs),
                        tuple(tuple(i) for i in ix))

            init = (tuple(tuple(neg_inf for _ in range(K))
                          for _ in range(ncc)),
                    tuple(tuple(zero_i for _ in range(K))
                          for _ in range(ncc)))
            vss, ixs = lax.fori_loop(0, E, body, init, unroll=4)

            for j in range(ncc):
                vs, ix = vss[j], ixs[j]
                col = pl.ds((c0 + j) * L, L)
                mx = vs[0]
                es = [jnp.exp(v - mx) for v in vs]
                tot = es[0]
                for kk in range(1, K):
                    tot = tot + es[kk]
                inv = 1.0 / tot
                for kk in range(K):
                    wloc[kk, col] = es[kk] * inv
                    iloc[kk, col] = ix[kk]
            return 0

        lax.fori_loop(0, ncol // ncc, col_pair, 0)

        pltpu.sync_copy(wloc, w_hbm.at[bi, :, pl.ds(off, tpw)])
        pltpu.sync_copy(iloc, i_hbm.at[bi, :, pl.ds(off, tpw)])

    return topk


def kernel(x, W_router):
    b, s, h = x.shape
    wt = W_router.T
    topk = _make_topk_sc(1, s)
    lts, wts, its = [], [], []
    for bi in range(b):
        lt = _router_logits_t(x[bi], wt, 1, s)
        lts.append(lt)
    for bi in range(b):
        wt_c, it_c = topk(lts[bi])
        wts.append(wt_c)
        its.append(it_c)
    logitst = jnp.concatenate(lts, axis=0)
    weights_t = jnp.concatenate(wts, axis=0)
    indices_t = jnp.concatenate(its, axis=0)
    return (jnp.swapaxes(weights_t, 1, 2),
            jnp.swapaxes(indices_t, 1, 2),
            jnp.swapaxes(logitst, 1, 2))


# speedup vs baseline: 1.8986x; 1.8986x over previous
import functools

import jax
import jax.numpy as jnp
from jax import lax
from jax.experimental import pallas as pl
from jax.experimental.pallas import tpu as pltpu
from jax.experimental.pallas import tpu_sc as plsc

E = 64
K = 8
L = 16
NW = 32


def _logits_body(wt_ref, x_ref, outt_ref):
    outt_ref[0] = lax.dot_general(
        wt_ref[...], x_ref[...],
        dimension_numbers=(((1,), (1,)), ((), ())),
        preferred_element_type=jnp.float32,
    )


def _router_logits_t(x2d, wt, b, s):
    t, h = x2d.shape
    bt = 1024
    nblk = t // bt
    bpb = s // bt
    return pl.pallas_call(
        _logits_body,
        grid=(nblk,),
        in_specs=[
            pl.BlockSpec((E, h), lambda i: (0, 0)),
            pl.BlockSpec((bt, h), lambda i: (i, 0)),
        ],
        out_specs=pl.BlockSpec((1, E, bt), lambda i: (i // bpb, 0, i % bpb)),
        out_shape=jax.ShapeDtypeStruct((b, E, s), jnp.float32),
    )(wt, x2d)


def _make_topk_sc(b, s, interpret=False):
    nc, ns = 2, 16
    t = b * s
    tpw = t // NW
    ncol = tpw // L
    wpb = s // tpw

    mesh = plsc.VectorSubcoreMesh(core_axis_name="c", subcore_axis_name="s",
                                  num_cores=nc, num_subcores=ns)

    @functools.partial(
        pl.kernel,
        out_type=(
            jax.ShapeDtypeStruct((b, K, s), jnp.float32),
            jax.ShapeDtypeStruct((b, K, s), jnp.int32),
        ),
        mesh=mesh,
        scratch_types=[
            pltpu.VMEM((E, tpw), jnp.float32),
            pltpu.VMEM((K, tpw), jnp.float32),
            pltpu.VMEM((K, tpw), jnp.int32),
        ],
        compiler_params=pltpu.CompilerParams(needs_layout_passes=False),
        interpret=interpret,
    )
    def topk(logitst_hbm, w_hbm, i_hbm, loc, wloc, iloc):
        wid = lax.axis_index("s") * nc + lax.axis_index("c")
        bi = wid // wpb
        off = (wid % wpb) * tpw
        pltpu.sync_copy(logitst_hbm.at[bi, :, pl.ds(off, tpw)], loc)

        neg_inf = jnp.full((L,), -jnp.inf, jnp.float32)
        zero_i = jnp.zeros((L,), jnp.int32)
        ncc = 2

        def col_pair(cp, _):
            c0 = cp * ncc

            def body(e, carry):
                vs = [list(v) for v in carry[0]]
                ix = [list(i) for i in carry[1]]
                eidx = jnp.full((L,), e, jnp.int32)
                for j in range(ncc):
                    cur_v = loc[e, pl.ds((c0 + j) * L, L)]
                    cur_i = eidx
                    for kk in range(K):
                        m = cur_v > vs[j][kk]
                        nv = jnp.maximum(cur_v, vs[j][kk])
                        lo = jnp.minimum(cur_v, vs[j][kk])
                        ni = jnp.where(m, cur_i, ix[j][kk])
                        cur_i = jnp.where(m, ix[j][kk], cur_i)
                        vs[j][kk] = nv
                        ix[j][kk] = ni
                        cur_v = lo
                    del cur_v, cur_i
                return (tuple(tuple(v) for v in vs),
                        tuple(tuple(i) for i in ix))

            init = (tuple(tuple(neg_inf for _ in range(K))
                          for _ in range(ncc)),
                    tuple(tuple(zero_i for _ in range(K))
                          for _ in range(ncc)))
            vss, ixs = lax.fori_loop(0, E, body, init, unroll=8)

            for j in range(ncc):
                vs, ix = vss[j], ixs[j]
                col = pl.ds((c0 + j) * L, L)
                mx = vs[0]
                es = [jnp.exp(v - mx) for v in vs]
                tot = es[0]
                for kk in range(1, K):
                    tot = tot + es[kk]
                inv = 1.0 / tot
                for kk in range(K):
                    wloc[kk, col] = es[kk] * inv
                    iloc[kk, col] = ix[kk]
            return 0

        lax.fori_loop(0, ncol // ncc, col_pair, 0)

        pltpu.sync_copy(wloc, w_hbm.at[bi, :, pl.ds(off, tpw)])
        pltpu.sync_copy(iloc, i_hbm.at[bi, :, pl.ds(off, tpw)])

    return topk


def kernel(x, W_router):
    b, s, h = x.shape
    x2d = x.reshape(b * s, h)
    wt = W_router.T
    logitst = _router_logits_t(x2d, wt, b, s)
    weights_t, indices_t = _make_topk_sc(b, s)(logitst)
    return (jnp.swapaxes(weights_t, 1, 2),
            jnp.swapaxes(indices_t, 1, 2),
            jnp.swapaxes(logitst, 1, 2))


# speedup vs baseline: 1.8991x; 1.0002x over previous
import functools

import jax
import jax.numpy as jnp
from jax import lax
from jax.experimental import pallas as pl
from jax.experimental.pallas import tpu as pltpu
from jax.experimental.pallas import tpu_sc as plsc

E = 64
K = 8
L = 16
NW = 32


def _logits_body(wt_ref, x_ref, outt_ref):
    outt_ref[0] = lax.dot_general(
        wt_ref[...], x_ref[...],
        dimension_numbers=(((1,), (1,)), ((), ())),
        preferred_element_type=jnp.float32,
    )


def _router_logits_t(x2d, wt, b, s):
    t, h = x2d.shape
    bt = 1024
    nblk = t // bt
    bpb = s // bt
    return pl.pallas_call(
        _logits_body,
        grid=(nblk,),
        in_specs=[
            pl.BlockSpec((E, h), lambda i: (0, 0)),
            pl.BlockSpec((bt, h), lambda i: (i, 0)),
        ],
        out_specs=pl.BlockSpec((1, E, bt), lambda i: (i // bpb, 0, i % bpb)),
        out_shape=jax.ShapeDtypeStruct((b, E, s), jnp.float32),
    )(wt, x2d)


def _make_topk_sc(b, s, interpret=False):
    nc, ns = 2, 16
    t = b * s
    tpw = t // NW
    ncol = tpw // L
    wpb = s // tpw

    mesh = plsc.VectorSubcoreMesh(core_axis_name="c", subcore_axis_name="s",
                                  num_cores=nc, num_subcores=ns)

    @functools.partial(
        pl.kernel,
        out_type=(
            jax.ShapeDtypeStruct((b, K, s), jnp.float32),
            jax.ShapeDtypeStruct((b, K, s), jnp.int32),
        ),
        mesh=mesh,
        scratch_types=[
            pltpu.VMEM((E, tpw), jnp.float32),
            pltpu.VMEM((K, tpw), jnp.float32),
            pltpu.VMEM((K, tpw), jnp.int32),
        ],
        compiler_params=pltpu.CompilerParams(needs_layout_passes=False,
                                             disable_bounds_checks=True,
                                             skip_device_barrier=True),
        interpret=interpret,
    )
    def topk(logitst_hbm, w_hbm, i_hbm, loc, wloc, iloc):
        wid = lax.axis_index("s") * nc + lax.axis_index("c")
        bi = wid // wpb
        off = (wid % wpb) * tpw
        pltpu.sync_copy(logitst_hbm.at[bi, :, pl.ds(off, tpw)], loc)

        neg_inf = jnp.full((L,), -jnp.inf, jnp.float32)
        zero_i = jnp.zeros((L,), jnp.int32)
        ncc = 2

        def col_pair(cp, _):
            c0 = cp * ncc

            def body(e, carry):
                vs = [list(v) for v in carry[0]]
                ix = [list(i) for i in carry[1]]
                eidx = jnp.full((L,), e, jnp.int32)
                for j in range(ncc):
                    cur_v = loc[e, pl.ds((c0 + j) * L, L)]
                    cur_i = eidx
                    for kk in range(K):
                        m = cur_v > vs[j][kk]
                        nv = jnp.maximum(cur_v, vs[j][kk])
                        lo = jnp.minimum(cur_v, vs[j][kk])
                        ni = jnp.where(m, cur_i, ix[j][kk])
                        cur_i = jnp.where(m, ix[j][kk], cur_i)
                        vs[j][kk] = nv
                        ix[j][kk] = ni
                        cur_v = lo
                    del cur_v, cur_i
                return (tuple(tuple(v) for v in vs),
                        tuple(tuple(i) for i in ix))

            init = (tuple(tuple(neg_inf for _ in range(K))
                          for _ in range(ncc)),
                    tuple(tuple(zero_i for _ in range(K))
                          for _ in range(ncc)))
            vss, ixs = lax.fori_loop(0, E, body, init, unroll=8)

            for j in range(ncc):
                vs, ix = vss[j], ixs[j]
                col = pl.ds((c0 + j) * L, L)
                mx = vs[0]
                es = [jnp.exp(v - mx) for v in vs]
                tot = es[0]
                for kk in range(1, K):
                    tot = tot + es[kk]
                inv = 1.0 / tot
                for kk in range(K):
                    wloc[kk, col] = es[kk] * inv
                    iloc[kk, col] = ix[kk]
            return 0

        lax.fori_loop(0, ncol // ncc, col_pair, 0)

        pltpu.sync_copy(wloc, w_hbm.at[bi, :, pl.ds(off, tpw)])
        pltpu.sync_copy(iloc, i_hbm.at[bi, :, pl.ds(off, tpw)])

    return topk


def kernel(x, W_router):
    b, s, h = x.shape
    x2d = x.reshape(b * s, h)
    wt = W_router.T
    logitst = _router_logits_t(x2d, wt, b, s)
    weights_t, indices_t = _make_topk_sc(b, s)(logitst)
    return (jnp.swapaxes(weights_t, 1, 2),
            jnp.swapaxes(indices_t, 1, 2),
            jnp.swapaxes(logitst, 1, 2))
